# baseline (speedup 1.0000x reference)
import functools

import jax
import jax.numpy as jnp
from jax import lax
from jax.experimental import pallas as pl
from jax.experimental.pallas import tpu as pltpu, tpu_sc as plsc

N = 10000
NPAD = 10240
IN_D = 128
HID = 128
OUT_D = 64
E = 320000
NC = 2
NS = 16
NW = NC * NS
K = 128
EPW = 10240
EPAD = EPW * NW
EPWP = EPW + K
NPADH = NPAD // NC
NACC = NPADH + 8
RPT = NPADH // NS
LH = 16

_mesh = lambda: plsc.VectorSubcoreMesh(
    core_axis_name="c", subcore_axis_name="s", num_cores=NC, num_subcores=NS)


@functools.cache
def _make_prep():
    @functools.partial(
        pl.kernel,
        out_type=(
            jax.ShapeDtypeStruct((2, NW, NPAD), jnp.float32),
            jax.ShapeDtypeStruct((2, NW, EPWP), jnp.int32),
            jax.ShapeDtypeStruct((2, NW, EPWP), jnp.int32),
            jax.ShapeDtypeStruct((NW, 16), jnp.int32),
        ),
        mesh=_mesh(),
        compiler_params=pltpu.CompilerParams(needs_layout_passes=False),
        scratch_types=[
            pltpu.VMEM((EPW,), jnp.int32),
            pltpu.VMEM((EPW,), jnp.int32),
            pltpu.VMEM((NPAD,), jnp.float32),
            pltpu.VMEM((EPWP,), jnp.int32),
            pltpu.VMEM((EPWP,), jnp.int32),
            pltpu.VMEM((EPWP,), jnp.int32),
            pltpu.VMEM((EPWP,), jnp.int32),
            pltpu.VMEM((16,), jnp.int32),
        ],
    )
    def _prep_sc(src_hbm, dst_hbm, zh_hbm, hist_hbm, psrc_hbm, pdst_hbm,
                 cnt_hbm, si_all, di_all, hist_v, ls_v, ld_v, hs_v, hd_v,
                 cnt_v):
        cid = lax.axis_index("c")
        sid = lax.axis_index("s")
        wid = sid * NC + cid
        base = pl.multiple_of(wid * EPW, K)
        pltpu.sync_copy(src_hbm.at[pl.ds(base, EPW)], si_all)
        pltpu.sync_copy(dst_hbm.at[pl.ds(base, EPW)], di_all)
        ones = jnp.full((16,), 1.0, jnp.float32)

        def count(ref):
            def step(j, carry):
                plsc.addupdate_scatter(hist_v, [ref[pl.ds(j * 16, 16)]], ones)
                return carry
            return step

        pltpu.sync_copy(zh_hbm, hist_v)
        lax.fori_loop(0, EPW // 16, count(si_all), 0)
        pltpu.sync_copy(hist_v, hist_hbm.at[0, wid])
        pltpu.sync_copy(zh_hbm, hist_v)
        lax.fori_loop(0, EPW // 16, count(di_all), 0)
        pltpu.sync_copy(hist_v, hist_hbm.at[1, wid])

        def part(j, carry):
            nlo, nhi = carry
            sv = si_all[pl.ds(j * 16, 16)]
            dv = di_all[pl.ds(j * 16, 16)]
            m = dv < NPADH
            plsc.store_compressed(ls_v.at[pl.ds(nlo, 16)], sv, mask=m)
            plsc.store_compressed(ld_v.at[pl.ds(nlo, 16)], dv, mask=m)
            nm = jnp.logical_not(m)
            plsc.store_compressed(hs_v.at[pl.ds(nhi, 16)], sv, mask=nm)
            plsc.store_compressed(hd_v.at[pl.ds(nhi, 16)], dv, mask=nm)
            c = jnp.sum(m.astype(jnp.int32))
            return (nlo + c, nhi + 16 - c)

        nlo, nhi = lax.fori_loop(0, EPW // 16, part, (0, 0))

        dsrc = jnp.full((16,), N, jnp.int32)
        for j in range(K // 16):
            ls_v[pl.ds(nlo + j * 16, 16)] = dsrc
            ld_v[pl.ds(nlo + j * 16, 16)] = jnp.full((16,), NPADH, jnp.int32)
            hs_v[pl.ds(nhi + j * 16, 16)] = dsrc
            hd_v[pl.ds(nhi + j * 16, 16)] = jnp.full((16,), NPADH + NPADH,
                                                     jnp.int32)
        pltpu.sync_copy(ls_v, psrc_hbm.at[0, wid])
        pltpu.sync_copy(ld_v, pdst_hbm.at[0, wid])
        pltpu.sync_copy(hs_v, psrc_hbm.at[1, wid])
        pltpu.sync_copy(hd_v, pdst_hbm.at[1, wid])
        iot = lax.iota(jnp.int32, 16)
        nloc = (nlo + K - 1) // K
        nhic = (nhi + K - 1) // K
        cnt_v[...] = jnp.where(iot == 0, nloc, jnp.where(iot == 1, nhic, 0))
        pltpu.sync_copy(cnt_v, cnt_hbm.at[wid])

    return _prep_sc


def _make_agg(D):
    @functools.partial(
        pl.kernel,
        out_type=jax.ShapeDtypeStruct((NC, NPADH, D), jnp.float32),
        mesh=_mesh(),
        compiler_params=pltpu.CompilerParams(needs_layout_passes=False),
        scratch_types=[
            pltpu.VMEM((K,), jnp.int32),
            pltpu.VMEM((K,), jnp.int32),
            pltpu.VMEM((K,), jnp.int32),
            pltpu.VMEM((K, D), jnp.float32),
            pltpu.VMEM((RPT, D), jnp.float32),
            pltpu.VMEM((16,), jnp.int32),
            pltpu.SemaphoreType.DMA,
            pltpu.VMEM_SHARED((NACC, D), jnp.float32),
        ],
    )
    def _agg_sc(z_hbm, psrc_hbm, pdst_hbm, cnt_hbm, zd_hbm, out_hbm,
                si, di, di2, rows, buf, cnt_v, sem, acc):
        cid = lax.axis_index("c")
        sid = lax.axis_index("s")
        r0 = sid * RPT
        lo = cid * NPADH
        pltpu.sync_copy(zd_hbm.at[pl.ds(r0, RPT)], buf)
        pltpu.sync_copy(buf, acc.at[pl.ds(r0, RPT)])
        plsc.subcore_barrier()

        def chunk(r):
            def step(i, carry):
                base = i * K
                pltpu.sync_copy(psrc_hbm.at[cid, r, pl.ds(base, K)], si)
                pltpu.sync_copy(pdst_hbm.at[cid, r, pl.ds(base, K)], di)
                gth = pltpu.async_copy(z_hbm.at[si], rows, sem)
                for j in range(K // 16):
                    di2[pl.ds(j * 16, 16)] = di[pl.ds(j * 16, 16)] - lo
                gth.wait()
                pltpu.sync_copy(rows, acc.at[di2], add=True)
                return carry
            return step

        iot = lax.iota(jnp.int32, 16)
        for t in range(2):
            r = 2 * sid + t
            pltpu.sync_copy(cnt_hbm.at[r], cnt_v)
            nch = jnp.sum(jnp.where(iot == cid, cnt_v[...], 0))
            lax.fori_loop(0, nch, chunk(r), 0)
        plsc.subcore_barrier()
        pltpu.sync_copy(acc.at[pl.ds(r0, RPT)], buf)
        pltpu.sync_copy(buf, out_hbm.at[cid, pl.ds(r0, RPT)])

    return _agg_sc


_make_agg = functools.cache(_make_agg)


def _deg_norms(hp):
    degs = jnp.sum(hp[0], axis=0)
    degd = jnp.sum(hp[1], axis=0)
    ns = lax.rsqrt(jnp.maximum(degs, 1.0))
    nd = lax.rsqrt(jnp.maximum(degd, 1.0))
    return ns, nd


def _mm1_body(x_ref, w_ref, o_ref):
    o_ref[...] = jnp.dot(x_ref[...], w_ref[...],
                         preferred_element_type=jnp.float32)


def _scale_body(xw_ref, h_ref, o_ref):
    ns, _ = _deg_norms(h_ref[...])
    o_ref[...] = xw_ref[...] * ns[:, None]


def _mid_body(acc_ref, h_ref, b1_ref, g1_ref, be1_ref, w2_ref, o_ref):
    ns, nd = _deg_norms(h_ref[...])
    a = acc_ref[...]
    h = a * nd[:, None] + b1_ref[...][None, :]
    mask = (lax.broadcasted_iota(jnp.int32, (NPAD, 1), 0) < N).astype(jnp.float32)
    mean = jnp.sum(h * mask, axis=0) / N
    cen = h - mean[None, :]
    var = jnp.sum(cen * cen * mask, axis=0) / N
    hbn = cen * lax.rsqrt(var + 1e-5)[None, :] * g1_ref[...][None, :] + be1_ref[...][None, :]
    hr = jnp.maximum(hbn, 0.0)
    z2 = jnp.dot(hr, w2_ref[...], preferred_element_type=jnp.float32)
    o_ref[...] = z2 * ns[:, None] * mask


def _out_body(acc_ref, h_ref, b2_ref, g2_ref, be2_ref, o_ref):
    _, nd = _deg_norms(h_ref[...])
    a = acc_ref[...][:, :OUT_D]
    h = a * nd[:, None] + b2_ref[...][None, :]
    mask = (lax.broadcasted_iota(jnp.int32, (NPAD, 1), 0) < N).astype(jnp.float32)
    mean = jnp.sum(h * mask, axis=0) / N
    cen = h - mean[None, :]
    var = jnp.sum(cen * cen * mask, axis=0) / N
    hbn = cen * lax.rsqrt(var + 1e-5)[None, :] * g2_ref[...][None, :] + be2_ref[...][None, :]
    hh = hbn[:N, :]
    m = jnp.max(hh, axis=1, keepdims=True)
    ex = jnp.exp(hh - m)
    lse = jnp.log(jnp.sum(ex, axis=1, keepdims=True))
    o_ref[...] = hh - m - lse


def _tc_call(body, out_shape, *args):
    return pl.pallas_call(
        body, out_shape=jax.ShapeDtypeStruct(out_shape, jnp.float32))(*args)


def kernel(x, W1, b1, gamma1, beta1, W2, b2, gamma2, beta2, edge_index):
    src = edge_index[0].astype(jnp.int32)
    dst = edge_index[1].astype(jnp.int32)
    pad_e = EPAD - E
    srcp = jnp.concatenate([src, jnp.full((pad_e,), N, jnp.int32)])
    dstp = jnp.concatenate([dst, jnp.full((pad_e,), N, jnp.int32)])
    xp = jnp.pad(x, ((0, NPAD - N), (0, 0)))
    zh = jnp.zeros((NPAD,), jnp.float32)
    z128 = jnp.zeros((NPAD, HID), jnp.float32)
    W2p = jnp.pad(W2, ((0, 0), (0, HID - OUT_D)))

    histp, psrc, pdst, cnt = _make_prep()(srcp, dstp, zh)
    xw1 = _tc_call(_mm1_body, (NPAD, HID), xp, W1)
    z1 = _tc_call(_scale_body, (NPAD, HID), xw1, histp)
    agg = _make_agg(HID)
    acc1 = agg(z1, psrc, pdst, cnt, z128)
    z2 = _tc_call(_mid_body, (NPAD, HID), acc1.reshape(NPAD, HID), histp, b1,
                  gamma1, beta1, W2p)
    acc2 = agg(z2, psrc, pdst, cnt, z128)
    out = _tc_call(_out_body, (N, OUT_D), acc2.reshape(NPAD, HID), histp, b2,
                  gamma2, beta2)
    return out


# speedup vs baseline: 1.3912x; 1.3912x over previous
import functools

import jax
import jax.numpy as jnp
from jax import lax
from jax.experimental import pallas as pl
from jax.experimental.pallas import tpu as pltpu, tpu_sc as plsc

N = 10000
NPAD = 10240
IN_D = 128
HID = 128
OUT_D = 64
E = 320000
NC = 2
NS = 16
NW = NC * NS
K = 128
EPW = 10240
EPAD = EPW * NW
EPWP = EPW + K
NPADH = NPAD // NC
NACC = NPADH + 128
RPT = NPADH // NS
LH = 16

_mesh = lambda: plsc.VectorSubcoreMesh(
    core_axis_name="c", subcore_axis_name="s", num_cores=NC, num_subcores=NS)


@functools.cache
def _make_prep():
    @functools.partial(
        pl.kernel,
        out_type=(
            jax.ShapeDtypeStruct((2, NW, NPAD), jnp.float32),
            jax.ShapeDtypeStruct((2, NW, EPWP), jnp.int32),
            jax.ShapeDtypeStruct((2, NW, EPWP), jnp.int32),
            jax.ShapeDtypeStruct((NW, 16), jnp.int32),
        ),
        mesh=_mesh(),
        compiler_params=pltpu.CompilerParams(needs_layout_passes=False),
        scratch_types=[
            pltpu.VMEM((EPW,), jnp.int32),
            pltpu.VMEM((EPW,), jnp.int32),
            pltpu.VMEM((NPAD,), jnp.float32),
            pltpu.VMEM((EPWP,), jnp.int32),
            pltpu.VMEM((EPWP,), jnp.int32),
            pltpu.VMEM((EPWP,), jnp.int32),
            pltpu.VMEM((EPWP,), jnp.int32),
            pltpu.VMEM((16,), jnp.int32),
        ],
    )
    def _prep_sc(src_hbm, dst_hbm, zh_hbm, hist_hbm, psrc_hbm, pdst_hbm,
                 cnt_hbm, si_all, di_all, hist_v, ls_v, ld_v, hs_v, hd_v,
                 cnt_v):
        cid = lax.axis_index("c")
        sid = lax.axis_index("s")
        wid = sid * NC + cid
        base = pl.multiple_of(wid * EPW, K)
        pltpu.sync_copy(src_hbm.at[pl.ds(base, EPW)], si_all)
        pltpu.sync_copy(dst_hbm.at[pl.ds(base, EPW)], di_all)
        ones = jnp.full((16,), 1.0, jnp.float32)

        def count(ref):
            def step(j, carry):
                plsc.addupdate_scatter(hist_v, [ref[pl.ds(j * 16, 16)]], ones)
                return carry
            return step

        pltpu.sync_copy(zh_hbm, hist_v)
        lax.fori_loop(0, EPW // 16, count(si_all), 0)
        pltpu.sync_copy(hist_v, hist_hbm.at[0, wid])
        pltpu.sync_copy(zh_hbm, hist_v)
        lax.fori_loop(0, EPW // 16, count(di_all), 0)
        pltpu.sync_copy(hist_v, hist_hbm.at[1, wid])

        def part(j, carry):
            nlo, nhi = carry
            sv = si_all[pl.ds(j * 16, 16)]
            dv = di_all[pl.ds(j * 16, 16)]
            m = dv < NPADH
            plsc.store_compressed(ls_v.at[pl.ds(nlo, 16)], sv, mask=m)
            plsc.store_compressed(ld_v.at[pl.ds(nlo, 16)], dv, mask=m)
            nm = jnp.logical_not(m)
            plsc.store_compressed(hs_v.at[pl.ds(nhi, 16)], sv, mask=nm)
            plsc.store_compressed(hd_v.at[pl.ds(nhi, 16)], dv, mask=nm)
            c = jnp.sum(m.astype(jnp.int32))
            return (nlo + c, nhi + 16 - c)

        nlo, nhi = lax.fori_loop(0, EPW // 16, part, (0, 0))

        dsrc = jnp.full((16,), N, jnp.int32)
        iot16 = lax.iota(jnp.int32, 16)
        for j in range(K // 16):
            dummy = iot16 + (j * 16)
            ls_v[pl.ds(nlo + j * 16, 16)] = dsrc
            ld_v[pl.ds(nlo + j * 16, 16)] = NPADH + dummy
            hs_v[pl.ds(nhi + j * 16, 16)] = dsrc
            hd_v[pl.ds(nhi + j * 16, 16)] = 2 * NPADH + dummy
        pltpu.sync_copy(ls_v, psrc_hbm.at[0, wid])
        pltpu.sync_copy(ld_v, pdst_hbm.at[0, wid])
        pltpu.sync_copy(hs_v, psrc_hbm.at[1, wid])
        pltpu.sync_copy(hd_v, pdst_hbm.at[1, wid])
        iot = lax.iota(jnp.int32, 16)
        nloc = (nlo + K - 1) // K
        nhic = (nhi + K - 1) // K
        cnt_v[...] = jnp.where(iot == 0, nloc, jnp.where(iot == 1, nhic, 0))
        pltpu.sync_copy(cnt_v, cnt_hbm.at[wid])

    return _prep_sc


def _make_agg(D):
    @functools.partial(
        pl.kernel,
        out_type=jax.ShapeDtypeStruct((NC, NPADH, D), jnp.float32),
        mesh=_mesh(),
        compiler_params=pltpu.CompilerParams(needs_layout_passes=False),
        scratch_types=[
            pltpu.VMEM((K,), jnp.int32),
            pltpu.VMEM((K,), jnp.int32),
            pltpu.VMEM((K,), jnp.int32),
            pltpu.VMEM((K, D), jnp.float32),
            pltpu.VMEM((RPT, D), jnp.float32),
            pltpu.VMEM((16,), jnp.int32),
            pltpu.SemaphoreType.DMA,
            pltpu.VMEM_SHARED((NACC, D), jnp.float32),
        ],
    )
    def _agg_sc(z_hbm, psrc_hbm, pdst_hbm, cnt_hbm, zd_hbm, out_hbm,
                si, di, di2, rows, buf, cnt_v, sem, acc):
        cid = lax.axis_index("c")
        sid = lax.axis_index("s")
        r0 = sid * RPT
        lo = cid * NPADH
        pltpu.sync_copy(zd_hbm.at[pl.ds(r0, RPT)], buf)
        pltpu.sync_copy(buf, acc.at[pl.ds(r0, RPT)])
        plsc.subcore_barrier()

        def chunk(r):
            def step(i, carry):
                base = i * K
                pltpu.sync_copy(psrc_hbm.at[cid, r, pl.ds(base, K)], si)
                pltpu.sync_copy(pdst_hbm.at[cid, r, pl.ds(base, K)], di)
                gth = pltpu.async_copy(z_hbm.at[si], rows, sem)
                for j in range(K // 16):
                    di2[pl.ds(j * 16, 16)] = di[pl.ds(j * 16, 16)] - lo
                gth.wait()
                pltpu.sync_copy(rows, acc.at[di2], add=True)
                return carry
            return step

        iot = lax.iota(jnp.int32, 16)
        for t in range(2):
            r = 2 * sid + t
            pltpu.sync_copy(cnt_hbm.at[r], cnt_v)
            nch = jnp.sum(jnp.where(iot == cid, cnt_v[...], 0))
            lax.fori_loop(0, nch, chunk(r), 0)
        plsc.subcore_barrier()
        pltpu.sync_copy(acc.at[pl.ds(r0, RPT)], buf)
        pltpu.sync_copy(buf, out_hbm.at[cid, pl.ds(r0, RPT)])

    return _agg_sc


_make_agg = functools.cache(_make_agg)


def _deg_norms(hp):
    degs = jnp.sum(hp[0], axis=0)
    degd = jnp.sum(hp[1], axis=0)
    ns = lax.rsqrt(jnp.maximum(degs, 1.0))
    nd = lax.rsqrt(jnp.maximum(degd, 1.0))
    return ns, nd


def _mm1_body(x_ref, w_ref, o_ref):
    o_ref[...] = jnp.dot(x_ref[...], w_ref[...],
                         preferred_element_type=jnp.float32)


def _scale_body(xw_ref, h_ref, o_ref):
    ns, _ = _deg_norms(h_ref[...])
    o_ref[...] = xw_ref[...] * ns[:, None]


def _mid_body(acc_ref, h_ref, b1_ref, g1_ref, be1_ref, w2_ref, o_ref):
    ns, nd = _deg_norms(h_ref[...])
    a = acc_ref[...]
    h = a * nd[:, None] + b1_ref[...][None, :]
    mask = (lax.broadcasted_iota(jnp.int32, (NPAD, 1), 0) < N).astype(jnp.float32)
    mean = jnp.sum(h * mask, axis=0) / N
    cen = h - mean[None, :]
    var = jnp.sum(cen * cen * mask, axis=0) / N
    hbn = cen * lax.rsqrt(var + 1e-5)[None, :] * g1_ref[...][None, :] + be1_ref[...][None, :]
    hr = jnp.maximum(hbn, 0.0)
    z2 = jnp.dot(hr, w2_ref[...], preferred_element_type=jnp.float32)
    o_ref[...] = z2 * ns[:, None] * mask


def _out_body(acc_ref, h_ref, b2_ref, g2_ref, be2_ref, o_ref):
    _, nd = _deg_norms(h_ref[...])
    a = acc_ref[...][:, :OUT_D]
    h = a * nd[:, None] + b2_ref[...][None, :]
    mask = (lax.broadcasted_iota(jnp.int32, (NPAD, 1), 0) < N).astype(jnp.float32)
    mean = jnp.sum(h * mask, axis=0) / N
    cen = h - mean[None, :]
    var = jnp.sum(cen * cen * mask, axis=0) / N
    hbn = cen * lax.rsqrt(var + 1e-5)[None, :] * g2_ref[...][None, :] + be2_ref[...][None, :]
    hh = hbn[:N, :]
    m = jnp.max(hh, axis=1, keepdims=True)
    ex = jnp.exp(hh - m)
    lse = jnp.log(jnp.sum(ex, axis=1, keepdims=True))
    o_ref[...] = hh - m - lse


def _tc_call(body, out_shape, *args):
    return pl.pallas_call(
        body, out_shape=jax.ShapeDtypeStruct(out_shape, jnp.float32))(*args)


def kernel(x, W1, b1, gamma1, beta1, W2, b2, gamma2, beta2, edge_index):
    src = edge_index[0].astype(jnp.int32)
    dst = edge_index[1].astype(jnp.int32)
    ppt = EPW - E // NW
    pad_src = jnp.full((NW, ppt), N, jnp.int32)
    pad_dst = jnp.broadcast_to(N + jnp.arange(ppt, dtype=jnp.int32), (NW, ppt))
    srcp = jnp.concatenate([src.reshape(NW, -1), pad_src], axis=1).reshape(-1)
    dstp = jnp.concatenate([dst.reshape(NW, -1), pad_dst], axis=1).reshape(-1)
    xp = jnp.pad(x, ((0, NPAD - N), (0, 0)))
    zh = jnp.zeros((NPAD,), jnp.float32)
    z128 = jnp.zeros((NPAD, HID), jnp.float32)
    W2p = jnp.pad(W2, ((0, 0), (0, HID - OUT_D)))

    histp, psrc, pdst, cnt = _make_prep()(srcp, dstp, zh)
    xw1 = _tc_call(_mm1_body, (NPAD, HID), xp, W1)
    z1 = _tc_call(_scale_body, (NPAD, HID), xw1, histp)
    agg = _make_agg(HID)
    acc1 = agg(z1, psrc, pdst, cnt, z128)
    z2 = _tc_call(_mid_body, (NPAD, HID), acc1.reshape(NPAD, HID), histp, b1,
                  gamma1, beta1, W2p)
    acc2 = agg(z2, psrc, pdst, cnt, z128)
    out = _tc_call(_out_body, (N, OUT_D), acc2.reshape(NPAD, HID), histp, b2,
                  gamma2, beta2)
    return out


# speedup vs baseline: 3.1843x; 2.2889x over previous
import functools

import jax
import jax.numpy as jnp
from jax import lax
from jax.experimental import pallas as pl
from jax.experimental.pallas import tpu as pltpu, tpu_sc as plsc

N = 10000
NPAD = 10240
IN_D = 128
HID = 128
OUT_D = 64
E = 320000
NC = 2
NS = 16
NW = NC * NS
K = 128
EPW = 10240
EPAD = EPW * NW
EPWP = EPW + K
NPADH = NPAD // NC
NACC = NPADH + 128
RPT = NPADH // NS
LH = 16

_mesh = lambda: plsc.VectorSubcoreMesh(
    core_axis_name="c", subcore_axis_name="s", num_cores=NC, num_subcores=NS)


@functools.cache
def _make_prep():
    @functools.partial(
        pl.kernel,
        out_type=(
            jax.ShapeDtypeStruct((2, NW, NPAD), jnp.float32),
            jax.ShapeDtypeStruct((2, NW, EPWP), jnp.int32),
            jax.ShapeDtypeStruct((2, NW, EPWP), jnp.int32),
            jax.ShapeDtypeStruct((NW, 16), jnp.int32),
        ),
        mesh=_mesh(),
        compiler_params=pltpu.CompilerParams(needs_layout_passes=False),
        scratch_types=[
            pltpu.VMEM((EPW,), jnp.int32),
            pltpu.VMEM((EPW,), jnp.int32),
            pltpu.VMEM((NPAD,), jnp.float32),
            pltpu.VMEM((EPWP,), jnp.int32),
            pltpu.VMEM((EPWP,), jnp.int32),
            pltpu.VMEM((EPWP,), jnp.int32),
            pltpu.VMEM((EPWP,), jnp.int32),
            pltpu.VMEM((16,), jnp.int32),
        ],
    )
    def _prep_sc(src_hbm, dst_hbm, zh_hbm, hist_hbm, psrc_hbm, pdst_hbm,
                 cnt_hbm, si_all, di_all, hist_v, ls_v, ld_v, hs_v, hd_v,
                 cnt_v):
        cid = lax.axis_index("c")
        sid = lax.axis_index("s")
        wid = sid * NC + cid
        base = pl.multiple_of(wid * EPW, K)
        pltpu.sync_copy(src_hbm.at[pl.ds(base, EPW)], si_all)
        pltpu.sync_copy(dst_hbm.at[pl.ds(base, EPW)], di_all)
        ones = jnp.full((16,), 1.0, jnp.float32)

        def count(ref):
            def step(j, carry):
                plsc.addupdate_scatter(hist_v, [ref[pl.ds(j * 16, 16)]], ones)
                return carry
            return step

        pltpu.sync_copy(zh_hbm, hist_v)
        lax.fori_loop(0, EPW // 16, count(si_all), 0)
        pltpu.sync_copy(hist_v, hist_hbm.at[0, wid])
        pltpu.sync_copy(zh_hbm, hist_v)
        lax.fori_loop(0, EPW // 16, count(di_all), 0)
        pltpu.sync_copy(hist_v, hist_hbm.at[1, wid])

        def part(j, carry):
            nlo, nhi = carry
            sv = si_all[pl.ds(j * 16, 16)]
            dv = di_all[pl.ds(j * 16, 16)]
            m = dv < NPADH
            plsc.store_compressed(ls_v.at[pl.ds(nlo, 16)], sv, mask=m)
            plsc.store_compressed(ld_v.at[pl.ds(nlo, 16)], dv, mask=m)
            nm = jnp.logical_not(m)
            plsc.store_compressed(hs_v.at[pl.ds(nhi, 16)], sv, mask=nm)
            plsc.store_compressed(hd_v.at[pl.ds(nhi, 16)], dv, mask=nm)
            c = jnp.sum(m.astype(jnp.int32))
            return (nlo + c, nhi + 16 - c)

        nlo, nhi = lax.fori_loop(0, EPW // 16, part, (0, 0))

        iot16 = lax.iota(jnp.int32, 16)
        for j in range(K // 16):
            dummy = iot16 + (j * 16)
            dsrc = N + dummy
            ls_v[pl.ds(nlo + j * 16, 16)] = dsrc
            ld_v[pl.ds(nlo + j * 16, 16)] = NPADH + dummy
            hs_v[pl.ds(nhi + j * 16, 16)] = dsrc
            hd_v[pl.ds(nhi + j * 16, 16)] = 2 * NPADH + dummy
        pltpu.sync_copy(ls_v, psrc_hbm.at[0, wid])
        pltpu.sync_copy(ld_v, pdst_hbm.at[0, wid])
        pltpu.sync_copy(hs_v, psrc_hbm.at[1, wid])
        pltpu.sync_copy(hd_v, pdst_hbm.at[1, wid])
        iot = lax.iota(jnp.int32, 16)
        nloc = (nlo + K - 1) // K
        nhic = (nhi + K - 1) // K
        cnt_v[...] = jnp.where(iot == 0, nloc, jnp.where(iot == 1, nhic, 0))
        pltpu.sync_copy(cnt_v, cnt_hbm.at[wid])

    return _prep_sc


def _make_agg(D):
    @functools.partial(
        pl.kernel,
        out_type=jax.ShapeDtypeStruct((NC, NPADH, D), jnp.float32),
        mesh=_mesh(),
        compiler_params=pltpu.CompilerParams(needs_layout_passes=False),
        scratch_types=[
            pltpu.VMEM((K,), jnp.int32),
            pltpu.VMEM((K,), jnp.int32),
            pltpu.VMEM((K,), jnp.int32),
            pltpu.VMEM((K, D), jnp.float32),
            pltpu.VMEM((RPT, D), jnp.float32),
            pltpu.VMEM((16,), jnp.int32),
            pltpu.SemaphoreType.DMA,
            pltpu.VMEM_SHARED((NACC, D), jnp.float32),
        ],
    )
    def _agg_sc(z_hbm, psrc_hbm, pdst_hbm, cnt_hbm, zd_hbm, out_hbm,
                si, di, di2, rows, buf, cnt_v, sem, acc):
        cid = lax.axis_index("c")
        sid = lax.axis_index("s")
        r0 = sid * RPT
        lo = cid * NPADH
        pltpu.sync_copy(zd_hbm.at[pl.ds(r0, RPT)], buf)
        pltpu.sync_copy(buf, acc.at[pl.ds(r0, RPT)])
        plsc.subcore_barrier()

        def chunk(r):
            def step(i, carry):
                base = i * K
                pltpu.sync_copy(psrc_hbm.at[cid, r, pl.ds(base, K)], si)
                pltpu.sync_copy(pdst_hbm.at[cid, r, pl.ds(base, K)], di)
                gth = pltpu.async_copy(z_hbm.at[si], rows, sem)
                for j in range(K // 16):
                    di2[pl.ds(j * 16, 16)] = di[pl.ds(j * 16, 16)] - lo
                gth.wait()
                pltpu.sync_copy(rows, acc.at[di2], add=True)
                return carry
            return step

        iot = lax.iota(jnp.int32, 16)
        for t in range(2):
            r = 2 * sid + t
            pltpu.sync_copy(cnt_hbm.at[r], cnt_v)
            nch = jnp.sum(jnp.where(iot == cid, cnt_v[...], 0))
            lax.fori_loop(0, nch, chunk(r), 0)
        plsc.subcore_barrier()
        pltpu.sync_copy(acc.at[pl.ds(r0, RPT)], buf)
        pltpu.sync_copy(buf, out_hbm.at[cid, pl.ds(r0, RPT)])

    return _agg_sc


_make_agg = functools.cache(_make_agg)


def _deg_norms(hp):
    degs = jnp.sum(hp[0], axis=0)
    degd = jnp.sum(hp[1], axis=0)
    ns = lax.rsqrt(jnp.maximum(degs, 1.0))
    nd = lax.rsqrt(jnp.maximum(degd, 1.0))
    return ns, nd


def _mm1_body(x_ref, w_ref, o_ref):
    o_ref[...] = jnp.dot(x_ref[...], w_ref[...],
                         preferred_element_type=jnp.float32)


def _scale_body(xw_ref, h_ref, o_ref):
    ns, _ = _deg_norms(h_ref[...])
    o_ref[...] = xw_ref[...] * ns[:, None]


def _mid_body(acc_ref, h_ref, b1_ref, g1_ref, be1_ref, w2_ref, o_ref):
    ns, nd = _deg_norms(h_ref[...])
    a = acc_ref[...]
    h = a * nd[:, None] + b1_ref[...][None, :]
    mask = (lax.broadcasted_iota(jnp.int32, (NPAD, 1), 0) < N).astype(jnp.float32)
    mean = jnp.sum(h * mask, axis=0) / N
    cen = h - mean[None, :]
    var = jnp.sum(cen * cen * mask, axis=0) / N
    hbn = cen * lax.rsqrt(var + 1e-5)[None, :] * g1_ref[...][None, :] + be1_ref[...][None, :]
    hr = jnp.maximum(hbn, 0.0)
    z2 = jnp.dot(hr, w2_ref[...], preferred_element_type=jnp.float32)
    o_ref[...] = z2 * ns[:, None] * mask


def _out_body(acc_ref, h_ref, b2_ref, g2_ref, be2_ref, o_ref):
    _, nd = _deg_norms(h_ref[...])
    a = acc_ref[...][:, :OUT_D]
    h = a * nd[:, None] + b2_ref[...][None, :]
    mask = (lax.broadcasted_iota(jnp.int32, (NPAD, 1), 0) < N).astype(jnp.float32)
    mean = jnp.sum(h * mask, axis=0) / N
    cen = h - mean[None, :]
    var = jnp.sum(cen * cen * mask, axis=0) / N
    hbn = cen * lax.rsqrt(var + 1e-5)[None, :] * g2_ref[...][None, :] + be2_ref[...][None, :]
    hh = hbn[:N, :]
    m = jnp.max(hh, axis=1, keepdims=True)
    ex = jnp.exp(hh - m)
    lse = jnp.log(jnp.sum(ex, axis=1, keepdims=True))
    o_ref[...] = hh - m - lse


def _tc_call(body, out_shape, *args):
    return pl.pallas_call(
        body, out_shape=jax.ShapeDtypeStruct(out_shape, jnp.float32))(*args)


def kernel(x, W1, b1, gamma1, beta1, W2, b2, gamma2, beta2, edge_index):
    src = edge_index[0].astype(jnp.int32)
    dst = edge_index[1].astype(jnp.int32)
    ppt = EPW - E // NW
    spread = jnp.broadcast_to(N + jnp.arange(ppt, dtype=jnp.int32), (NW, ppt))
    pad_src = spread
    pad_dst = spread
    srcp = jnp.concatenate([src.reshape(NW, -1), pad_src], axis=1).reshape(-1)
    dstp = jnp.concatenate([dst.reshape(NW, -1), pad_dst], axis=1).reshape(-1)
    xp = jnp.pad(x, ((0, NPAD - N), (0, 0)))
    zh = jnp.zeros((NPAD,), jnp.float32)
    z128 = jnp.zeros((NPAD, HID), jnp.float32)
    W2p = jnp.pad(W2, ((0, 0), (0, HID - OUT_D)))

    histp, psrc, pdst, cnt = _make_prep()(srcp, dstp, zh)
    xw1 = _tc_call(_mm1_body, (NPAD, HID), xp, W1)
    z1 = _tc_call(_scale_body, (NPAD, HID), xw1, histp)
    agg = _make_agg(HID)
    acc1 = agg(z1, psrc, pdst, cnt, z128)
    z2 = _tc_call(_mid_body, (NPAD, HID), acc1.reshape(NPAD, HID), histp, b1,
                  gamma1, beta1, W2p)
    acc2 = agg(z2, psrc, pdst, cnt, z128)
    out = _tc_call(_out_body, (N, OUT_D), acc2.reshape(NPAD, HID), histp, b2,
                  gamma2, beta2)
    return out


# speedup vs baseline: 3.7891x; 1.1899x over previous
import functools

import jax
import jax.numpy as jnp
from jax import lax
from jax.experimental import pallas as pl
from jax.experimental.pallas import tpu as pltpu, tpu_sc as plsc

N = 10000
NPAD = 10240
IN_D = 128
HID = 128
OUT_D = 64
E = 320000
NC = 2
NS = 16
NW = NC * NS
K = 128
EPW = 10240
EPAD = EPW * NW
EPWP = EPW + K
NPADH = NPAD // NC
NACC = NPADH + 128
RPT = NPADH // NS
LH = 16

_mesh = lambda: plsc.VectorSubcoreMesh(
    core_axis_name="c", subcore_axis_name="s", num_cores=NC, num_subcores=NS)


@functools.cache
def _make_prep():
    @functools.partial(
        pl.kernel,
        out_type=(
            jax.ShapeDtypeStruct((2, NW, NPAD), jnp.float32),
            jax.ShapeDtypeStruct((2, NW, EPWP), jnp.int32),
            jax.ShapeDtypeStruct((2, NW, EPWP), jnp.int32),
            jax.ShapeDtypeStruct((NW, 16), jnp.int32),
        ),
        mesh=_mesh(),
        compiler_params=pltpu.CompilerParams(needs_layout_passes=False),
        scratch_types=[
            pltpu.VMEM((EPW,), jnp.int32),
            pltpu.VMEM((EPW,), jnp.int32),
            pltpu.VMEM((NPAD,), jnp.float32),
            pltpu.VMEM((EPWP,), jnp.int32),
            pltpu.VMEM((EPWP,), jnp.int32),
            pltpu.VMEM((EPWP,), jnp.int32),
            pltpu.VMEM((EPWP,), jnp.int32),
            pltpu.VMEM((16,), jnp.int32),
        ],
    )
    def _prep_sc(src_hbm, dst_hbm, zh_hbm, hist_hbm, psrc_hbm, pdst_hbm,
                 cnt_hbm, si_all, di_all, hist_v, ls_v, ld_v, hs_v, hd_v,
                 cnt_v):
        cid = lax.axis_index("c")
        sid = lax.axis_index("s")
        wid = sid * NC + cid
        base = pl.multiple_of(wid * EPW, K)
        pltpu.sync_copy(src_hbm.at[pl.ds(base, EPW)], si_all)
        pltpu.sync_copy(dst_hbm.at[pl.ds(base, EPW)], di_all)
        ones = jnp.full((16,), 1.0, jnp.float32)

        def count(ref):
            def step(j, carry):
                plsc.addupdate_scatter(hist_v, [ref[pl.ds(j * 16, 16)]], ones)
                return carry
            return step

        pltpu.sync_copy(zh_hbm, hist_v)
        lax.fori_loop(0, EPW // 16, count(si_all), 0)
        pltpu.sync_copy(hist_v, hist_hbm.at[0, wid])
        pltpu.sync_copy(zh_hbm, hist_v)
        lax.fori_loop(0, EPW // 16, count(di_all), 0)
        pltpu.sync_copy(hist_v, hist_hbm.at[1, wid])

        def part(j, carry):
            nlo, nhi = carry
            sv = si_all[pl.ds(j * 16, 16)]
            dv = di_all[pl.ds(j * 16, 16)]
            m = dv < NPADH
            plsc.store_compressed(ls_v.at[pl.ds(nlo, 16)], sv, mask=m)
            plsc.store_compressed(ld_v.at[pl.ds(nlo, 16)], dv, mask=m)
            nm = jnp.logical_not(m)
            plsc.store_compressed(hs_v.at[pl.ds(nhi, 16)], sv, mask=nm)
            plsc.store_compressed(hd_v.at[pl.ds(nhi, 16)], dv, mask=nm)
            c = jnp.sum(m.astype(jnp.int32))
            return (nlo + c, nhi + 16 - c)

        nlo, nhi = lax.fori_loop(0, EPW // 16, part, (0, 0))

        iot16 = lax.iota(jnp.int32, 16)
        for j in range(K // 16):
            dummy = iot16 + (j * 16)
            dsrc = N + dummy
            ls_v[pl.ds(nlo + j * 16, 16)] = dsrc
            ld_v[pl.ds(nlo + j * 16, 16)] = NPADH + dummy
            hs_v[pl.ds(nhi + j * 16, 16)] = dsrc
            hd_v[pl.ds(nhi + j * 16, 16)] = 2 * NPADH + dummy
        pltpu.sync_copy(ls_v, psrc_hbm.at[0, wid])
        pltpu.sync_copy(ld_v, pdst_hbm.at[0, wid])
        pltpu.sync_copy(hs_v, psrc_hbm.at[1, wid])
        pltpu.sync_copy(hd_v, pdst_hbm.at[1, wid])
        iot = lax.iota(jnp.int32, 16)
        nloc = (nlo + K - 1) // K
        nhic = (nhi + K - 1) // K
        cnt_v[...] = jnp.where(iot == 0, nloc, jnp.where(iot == 1, nhic, 0))
        pltpu.sync_copy(cnt_v, cnt_hbm.at[wid])

    return _prep_sc


def _make_agg(D):
    @functools.partial(
        pl.kernel,
        out_type=jax.ShapeDtypeStruct((NC, NPADH, D), jnp.float32),
        mesh=_mesh(),
        compiler_params=pltpu.CompilerParams(needs_layout_passes=False),
        scratch_types=[
            pltpu.VMEM((K,), jnp.int32),
            pltpu.VMEM((K,), jnp.int32),
            pltpu.VMEM((K,), jnp.int32),
            pltpu.VMEM((K,), jnp.int32),
            pltpu.VMEM((K,), jnp.int32),
            pltpu.VMEM((K,), jnp.int32),
            pltpu.VMEM((K, D), jnp.float32),
            pltpu.VMEM((K, D), jnp.float32),
            pltpu.VMEM((RPT, D), jnp.float32),
            pltpu.VMEM((16,), jnp.int32),
            pltpu.SemaphoreType.DMA,
            pltpu.SemaphoreType.DMA,
            pltpu.VMEM_SHARED((NACC, D), jnp.float32),
        ],
    )
    def _agg_sc(z_hbm, psrc_hbm, pdst_hbm, cnt_hbm, zd_hbm, out_hbm,
                si0, si1, di0, di1, dd0, dd1, rows0, rows1, buf, cnt_v,
                sem0, sem1, acc):
        si = [si0, si1]
        di = [di0, di1]
        dd = [dd0, dd1]
        rows = [rows0, rows1]
        sem = [sem0, sem1]
        cid = lax.axis_index("c")
        sid = lax.axis_index("s")
        r0 = sid * RPT
        lo = cid * NPADH
        pltpu.sync_copy(zd_hbm.at[pl.ds(r0, RPT)], buf)
        pltpu.sync_copy(buf, acc.at[pl.ds(r0, RPT)])
        plsc.subcore_barrier()

        def region(r):
            pltpu.sync_copy(cnt_hbm.at[r], cnt_v)
            iot = lax.iota(jnp.int32, 16)
            nch = jnp.sum(jnp.where(iot == cid, cnt_v[...], 0))

            def load_idx(i, b):
                base = i * K
                pltpu.sync_copy(psrc_hbm.at[cid, r, pl.ds(base, K)], si[b])
                pltpu.sync_copy(pdst_hbm.at[cid, r, pl.ds(base, K)], di[b])

            def gather(b):
                pltpu.async_copy(z_hbm.at[si[b]], rows[b], sem[b])

            def wait_gather(b):
                pltpu.make_async_copy(z_hbm.at[si[b]], rows[b], sem[b]).wait()

            def consume(b):
                for j in range(K // 16):
                    dd[b][pl.ds(j * 16, 16)] = di[b][pl.ds(j * 16, 16)] - lo
                pltpu.sync_copy(rows[b], acc.at[dd[b]], add=True)

            @pl.when(nch > 0)
            def _():
                load_idx(0, 0)
                gather(0)

            def body(h, carry):
                i0 = 2 * h
                i1 = i0 + 1
                i2 = i0 + 2

                @pl.when(i0 < nch)
                def _():
                    wait_gather(0)
                    @pl.when(i1 < nch)
                    def _():
                        load_idx(i1, 1)
                        gather(1)
                    consume(0)

                @pl.when(i1 < nch)
                def _():
                    wait_gather(1)
                    @pl.when(i2 < nch)
                    def _():
                        load_idx(i2, 0)
                        gather(0)
                    consume(1)
                return carry

            lax.fori_loop(0, (nch + 1) // 2, body, 0)

        for t in range(2):
            region(2 * sid + t)
        plsc.subcore_barrier()
        pltpu.sync_copy(acc.at[pl.ds(r0, RPT)], buf)
        pltpu.sync_copy(buf, out_hbm.at[cid, pl.ds(r0, RPT)])

    return _agg_sc


_make_agg = functools.cache(_make_agg)


def _deg_norms(hp):
    degs = jnp.sum(hp[0], axis=0)
    degd = jnp.sum(hp[1], axis=0)
    ns = lax.rsqrt(jnp.maximum(degs, 1.0))
    nd = lax.rsqrt(jnp.maximum(degd, 1.0))
    return ns, nd


def _mm1_body(x_ref, w_ref, o_ref):
    o_ref[...] = jnp.dot(x_ref[...], w_ref[...],
                         preferred_element_type=jnp.float32)


def _scale_body(xw_ref, h_ref, o_ref):
    ns, _ = _deg_norms(h_ref[...])
    o_ref[...] = xw_ref[...] * ns[:, None]


def _mid_body(acc_ref, h_ref, b1_ref, g1_ref, be1_ref, w2_ref, o_ref):
    ns, nd = _deg_norms(h_ref[...])
    a = acc_ref[...]
    h = a * nd[:, None] + b1_ref[...][None, :]
    mask = (lax.broadcasted_iota(jnp.int32, (NPAD, 1), 0) < N).astype(jnp.float32)
    mean = jnp.sum(h * mask, axis=0) / N
    cen = h - mean[None, :]
    var = jnp.sum(cen * cen * mask, axis=0) / N
    hbn = cen * lax.rsqrt(var + 1e-5)[None, :] * g1_ref[...][None, :] + be1_ref[...][None, :]
    hr = jnp.maximum(hbn, 0.0)
    z2 = jnp.dot(hr, w2_ref[...], preferred_element_type=jnp.float32)
    o_ref[...] = z2 * ns[:, None] * mask


def _out_body(acc_ref, h_ref, b2_ref, g2_ref, be2_ref, o_ref):
    _, nd = _deg_norms(h_ref[...])
    a = acc_ref[...][:, :OUT_D]
    h = a * nd[:, None] + b2_ref[...][None, :]
    mask = (lax.broadcasted_iota(jnp.int32, (NPAD, 1), 0) < N).astype(jnp.float32)
    mean = jnp.sum(h * mask, axis=0) / N
    cen = h - mean[None, :]
    var = jnp.sum(cen * cen * mask, axis=0) / N
    hbn = cen * lax.rsqrt(var + 1e-5)[None, :] * g2_ref[...][None, :] + be2_ref[...][None, :]
    hh = hbn[:N, :]
    m = jnp.max(hh, axis=1, keepdims=True)
    ex = jnp.exp(hh - m)
    lse = jnp.log(jnp.sum(ex, axis=1, keepdims=True))
    o_ref[...] = hh - m - lse


def _tc_call(body, out_shape, *args):
    return pl.pallas_call(
        body, out_shape=jax.ShapeDtypeStruct(out_shape, jnp.float32))(*args)


def kernel(x, W1, b1, gamma1, beta1, W2, b2, gamma2, beta2, edge_index):
    src = edge_index[0].astype(jnp.int32)
    dst = edge_index[1].astype(jnp.int32)
    ppt = EPW - E // NW
    spread = jnp.broadcast_to(N + jnp.arange(ppt, dtype=jnp.int32), (NW, ppt))
    pad_src = spread
    pad_dst = spread
    srcp = jnp.concatenate([src.reshape(NW, -1), pad_src], axis=1).reshape(-1)
    dstp = jnp.concatenate([dst.reshape(NW, -1), pad_dst], axis=1).reshape(-1)
    xp = jnp.pad(x, ((0, NPAD - N), (0, 0)))
    zh = jnp.zeros((NPAD,), jnp.float32)
    z128 = jnp.zeros((NPAD, HID), jnp.float32)
    W2p = jnp.pad(W2, ((0, 0), (0, HID - OUT_D)))

    histp, psrc, pdst, cnt = _make_prep()(srcp, dstp, zh)
    xw1 = _tc_call(_mm1_body, (NPAD, HID), xp, W1)
    z1 = _tc_call(_scale_body, (NPAD, HID), xw1, histp)
    agg = _make_agg(HID)
    acc1 = agg(z1, psrc, pdst, cnt, z128)
    z2 = _tc_call(_mid_body, (NPAD, HID), acc1.reshape(NPAD, HID), histp, b1,
                  gamma1, beta1, W2p)
    acc2 = agg(z2, psrc, pdst, cnt, z128)
    out = _tc_call(_out_body, (N, OUT_D), acc2.reshape(NPAD, HID), histp, b2,
                  gamma2, beta2)
    return out


# speedup vs baseline: 3.8605x; 1.0188x over previous
import functools

import jax
import jax.numpy as jnp
from jax import lax
from jax.experimental import pallas as pl
from jax.experimental.pallas import tpu as pltpu, tpu_sc as plsc

N = 10000
NPAD = 10240
IN_D = 128
HID = 128
OUT_D = 64
E = 320000
NC = 2
NS = 16
NW = NC * NS
K = 128
EPW = 10240
EPAD = EPW * NW
EPWP = EPW + K
NPADH = NPAD // NC
NACC = NPADH + 128
RPT = NPADH // NS
LH = 16

_mesh = lambda: plsc.VectorSubcoreMesh(
    core_axis_name="c", subcore_axis_name="s", num_cores=NC, num_subcores=NS)


@functools.cache
def _make_prep():
    @functools.partial(
        pl.kernel,
        out_type=(
            jax.ShapeDtypeStruct((2, NW, NPAD), jnp.float32),
            jax.ShapeDtypeStruct((2, NW, EPWP), jnp.int32),
            jax.ShapeDtypeStruct((2, NW, EPWP), jnp.int32),
            jax.ShapeDtypeStruct((NW, 16), jnp.int32),
        ),
        mesh=_mesh(),
        compiler_params=pltpu.CompilerParams(needs_layout_passes=False),
        scratch_types=[
            pltpu.VMEM((EPW,), jnp.int32),
            pltpu.VMEM((EPW,), jnp.int32),
            pltpu.VMEM((NPAD,), jnp.float32),
            pltpu.VMEM((EPWP,), jnp.int32),
            pltpu.VMEM((EPWP,), jnp.int32),
            pltpu.VMEM((EPWP,), jnp.int32),
            pltpu.VMEM((EPWP,), jnp.int32),
            pltpu.VMEM((16,), jnp.int32),
        ],
    )
    def _prep_sc(src_hbm, dst_hbm, zh_hbm, hist_hbm, psrc_hbm, pdst_hbm,
                 cnt_hbm, si_all, di_all, hist_v, ls_v, ld_v, hs_v, hd_v,
                 cnt_v):
        cid = lax.axis_index("c")
        sid = lax.axis_index("s")
        wid = sid * NC + cid
        base = pl.multiple_of(wid * EPW, K)
        pltpu.sync_copy(src_hbm.at[pl.ds(base, EPW)], si_all)
        pltpu.sync_copy(dst_hbm.at[pl.ds(base, EPW)], di_all)
        ones = jnp.full((16,), 1.0, jnp.float32)

        def count(ref):
            def step(j, carry):
                plsc.addupdate_scatter(hist_v, [ref[pl.ds(j * 16, 16)]], ones)
                return carry
            return step

        pltpu.sync_copy(zh_hbm, hist_v)
        lax.fori_loop(0, EPW // 16, count(si_all), 0)
        pltpu.sync_copy(hist_v, hist_hbm.at[0, wid])
        pltpu.sync_copy(zh_hbm, hist_v)
        lax.fori_loop(0, EPW // 16, count(di_all), 0)
        pltpu.sync_copy(hist_v, hist_hbm.at[1, wid])

        def part(j, carry):
            nlo, nhi = carry
            sv = si_all[pl.ds(j * 16, 16)]
            dv = di_all[pl.ds(j * 16, 16)]
            m = dv < NPADH
            plsc.store_compressed(ls_v.at[pl.ds(nlo, 16)], sv, mask=m)
            plsc.store_compressed(ld_v.at[pl.ds(nlo, 16)], dv, mask=m)
            nm = jnp.logical_not(m)
            plsc.store_compressed(hs_v.at[pl.ds(nhi, 16)], sv, mask=nm)
            plsc.store_compressed(hd_v.at[pl.ds(nhi, 16)], dv, mask=nm)
            c = jnp.sum(m.astype(jnp.int32))
            return (nlo + c, nhi + 16 - c)

        nlo, nhi = lax.fori_loop(0, EPW // 16, part, (0, 0))

        iot16 = lax.iota(jnp.int32, 16)
        for j in range(K // 16):
            dummy = iot16 + (j * 16)
            dsrc = N + dummy
            ls_v[pl.ds(nlo + j * 16, 16)] = dsrc
            ld_v[pl.ds(nlo + j * 16, 16)] = NPADH + dummy
            hs_v[pl.ds(nhi + j * 16, 16)] = dsrc
            hd_v[pl.ds(nhi + j * 16, 16)] = 2 * NPADH + dummy
        pltpu.sync_copy(ls_v, psrc_hbm.at[0, wid])
        pltpu.sync_copy(ld_v, pdst_hbm.at[0, wid])
        pltpu.sync_copy(hs_v, psrc_hbm.at[1, wid])
        pltpu.sync_copy(hd_v, pdst_hbm.at[1, wid])
        iot = lax.iota(jnp.int32, 16)
        nloc = (nlo + K - 1) // K
        nhic = (nhi + K - 1) // K
        cnt_v[...] = jnp.where(iot == 0, nloc, jnp.where(iot == 1, nhic, 0))
        pltpu.sync_copy(cnt_v, cnt_hbm.at[wid])

    return _prep_sc


def _make_agg(D):
    @functools.partial(
        pl.kernel,
        out_type=jax.ShapeDtypeStruct((NC, NPADH, D), jnp.float32),
        mesh=_mesh(),
        compiler_params=pltpu.CompilerParams(needs_layout_passes=False),
        scratch_types=[
            pltpu.VMEM((K,), jnp.int32),
            pltpu.VMEM((K,), jnp.int32),
            pltpu.VMEM((K,), jnp.int32),
            pltpu.VMEM((K,), jnp.int32),
            pltpu.VMEM((K,), jnp.int32),
            pltpu.VMEM((K,), jnp.int32),
            pltpu.VMEM((K, D), jnp.float32),
            pltpu.VMEM((K, D), jnp.float32),
            pltpu.VMEM((RPT, D), jnp.float32),
            pltpu.VMEM((16,), jnp.int32),
            pltpu.SemaphoreType.DMA,
            pltpu.SemaphoreType.DMA,
            pltpu.SemaphoreType.DMA,
            pltpu.SemaphoreType.DMA,
            pltpu.VMEM_SHARED((NACC, D), jnp.float32),
        ],
    )
    def _agg_sc(z_hbm, psrc_hbm, pdst_hbm, cnt_hbm, zd_hbm, out_hbm,
                si0, si1, di0, di1, dd0, dd1, rows0, rows1, buf, cnt_v,
                sem0, sem1, sem0b, sem1b, acc):
        si = [si0, si1]
        di = [di0, di1]
        dd = [dd0, dd1]
        rows = [rows0, rows1]
        sem = [sem0, sem1]
        semb = [sem0b, sem1b]
        cid = lax.axis_index("c")
        sid = lax.axis_index("s")
        r0 = sid * RPT
        lo = cid * NPADH
        pltpu.sync_copy(zd_hbm.at[pl.ds(r0, RPT)], buf)
        pltpu.sync_copy(buf, acc.at[pl.ds(r0, RPT)])
        plsc.subcore_barrier()

        def region(r):
            pltpu.sync_copy(cnt_hbm.at[r], cnt_v)
            iot = lax.iota(jnp.int32, 16)
            nch = jnp.sum(jnp.where(iot == cid, cnt_v[...], 0))

            def load_idx(i, b):
                base = i * K
                pltpu.sync_copy(psrc_hbm.at[cid, r, pl.ds(base, K)], si[b])
                pltpu.sync_copy(pdst_hbm.at[cid, r, pl.ds(base, K)], di[b])

            H2 = K // 2

            def gather(b):
                pltpu.async_copy(z_hbm.at[si[b].at[pl.ds(0, H2)]],
                                 rows[b].at[pl.ds(0, H2)], sem[b])
                pltpu.async_copy(z_hbm.at[si[b].at[pl.ds(H2, H2)]],
                                 rows[b].at[pl.ds(H2, H2)], semb[b])

            def wait_gather(b):
                pltpu.make_async_copy(z_hbm.at[si[b].at[pl.ds(0, H2)]],
                                      rows[b].at[pl.ds(0, H2)], sem[b]).wait()
                pltpu.make_async_copy(z_hbm.at[si[b].at[pl.ds(H2, H2)]],
                                      rows[b].at[pl.ds(H2, H2)], semb[b]).wait()

            def consume(b):
                for j in range(K // 16):
                    dd[b][pl.ds(j * 16, 16)] = di[b][pl.ds(j * 16, 16)] - lo
                pltpu.sync_copy(rows[b], acc.at[dd[b]], add=True)

            @pl.when(nch > 0)
            def _():
                load_idx(0, 0)
                gather(0)

            def body(h, carry):
                i0 = 2 * h
                i1 = i0 + 1
                i2 = i0 + 2

                @pl.when(i0 < nch)
                def _():
                    wait_gather(0)
                    @pl.when(i1 < nch)
                    def _():
                        load_idx(i1, 1)
                        gather(1)
                    consume(0)

                @pl.when(i1 < nch)
                def _():
                    wait_gather(1)
                    @pl.when(i2 < nch)
                    def _():
                        load_idx(i2, 0)
                        gather(0)
                    consume(1)
                return carry

            lax.fori_loop(0, (nch + 1) // 2, body, 0)

        for t in range(2):
            region(2 * sid + t)
        plsc.subcore_barrier()
        pltpu.sync_copy(acc.at[pl.ds(r0, RPT)], buf)
        pltpu.sync_copy(buf, out_hbm.at[cid, pl.ds(r0, RPT)])

    return _agg_sc


_make_agg = functools.cache(_make_agg)


def _deg_norms(hp):
    degs = jnp.sum(hp[0], axis=0)
    degd = jnp.sum(hp[1], axis=0)
    ns = lax.rsqrt(jnp.maximum(degs, 1.0))
    nd = lax.rsqrt(jnp.maximum(degd, 1.0))
    return ns, nd


def _mm1_body(x_ref, w_ref, o_ref):
    o_ref[...] = jnp.dot(x_ref[...], w_ref[...],
                         preferred_element_type=jnp.float32)


def _scale_body(xw_ref, h_ref, o_ref):
    ns, _ = _deg_norms(h_ref[...])
    o_ref[...] = xw_ref[...] * ns[:, None]


def _mid_body(acc_ref, h_ref, b1_ref, g1_ref, be1_ref, w2_ref, o_ref):
    ns, nd = _deg_norms(h_ref[...])
    a = acc_ref[...]
    h = a * nd[:, None] + b1_ref[...][None, :]
    mask = (lax.broadcasted_iota(jnp.int32, (NPAD, 1), 0) < N).astype(jnp.float32)
    mean = jnp.sum(h * mask, axis=0) / N
    cen = h - mean[None, :]
    var = jnp.sum(cen * cen * mask, axis=0) / N
    hbn = cen * lax.rsqrt(var + 1e-5)[None, :] * g1_ref[...][None, :] + be1_ref[...][None, :]
    hr = jnp.maximum(hbn, 0.0)
    z2 = jnp.dot(hr, w2_ref[...], preferred_element_type=jnp.float32)
    o_ref[...] = z2 * ns[:, None] * mask


def _out_body(acc_ref, h_ref, b2_ref, g2_ref, be2_ref, o_ref):
    _, nd = _deg_norms(h_ref[...])
    a = acc_ref[...][:, :OUT_D]
    h = a * nd[:, None] + b2_ref[...][None, :]
    mask = (lax.broadcasted_iota(jnp.int32, (NPAD, 1), 0) < N).astype(jnp.float32)
    mean = jnp.sum(h * mask, axis=0) / N
    cen = h - mean[None, :]
    var = jnp.sum(cen * cen * mask, axis=0) / N
    hbn = cen * lax.rsqrt(var + 1e-5)[None, :] * g2_ref[...][None, :] + be2_ref[...][None, :]
    hh = hbn[:N, :]
    m = jnp.max(hh, axis=1, keepdims=True)
    ex = jnp.exp(hh - m)
    lse = jnp.log(jnp.sum(ex, axis=1, keepdims=True))
    o_ref[...] = hh - m - lse


def _tc_call(body, out_shape, *args):
    return pl.pallas_call(
        body, out_shape=jax.ShapeDtypeStruct(out_shape, jnp.float32))(*args)


def kernel(x, W1, b1, gamma1, beta1, W2, b2, gamma2, beta2, edge_index):
    src = edge_index[0].astype(jnp.int32)
    dst = edge_index[1].astype(jnp.int32)
    ppt = EPW - E // NW
    spread = jnp.broadcast_to(N + jnp.arange(ppt, dtype=jnp.int32), (NW, ppt))
    pad_src = spread
    pad_dst = spread
    srcp = jnp.concatenate([src.reshape(NW, -1), pad_src], axis=1).reshape(-1)
    dstp = jnp.concatenate([dst.reshape(NW, -1), pad_dst], axis=1).reshape(-1)
    xp = jnp.pad(x, ((0, NPAD - N), (0, 0)))
    zh = jnp.zeros((NPAD,), jnp.float32)
    z128 = jnp.zeros((NPAD, HID), jnp.float32)
    W2p = jnp.pad(W2, ((0, 0), (0, HID - OUT_D)))

    histp, psrc, pdst, cnt = _make_prep()(srcp, dstp, zh)
    xw1 = _tc_call(_mm1_body, (NPAD, HID), xp, W1)
    z1 = _tc_call(_scale_body, (NPAD, HID), xw1, histp)
    agg = _make_agg(HID)
    acc1 = agg(z1, psrc, pdst, cnt, z128)
    z2 = _tc_call(_mid_body, (NPAD, HID), acc1.reshape(NPAD, HID), histp, b1,
                  gamma1, beta1, W2p)
    acc2 = agg(z2, psrc, pdst, cnt, z128)
    out = _tc_call(_out_body, (N, OUT_D), acc2.reshape(NPAD, HID), histp, b2,
                  gamma2, beta2)
    return out


# speedup vs baseline: 3.9427x; 1.0213x over previous
import functools

import jax
import jax.numpy as jnp
from jax import lax
from jax.experimental import pallas as pl
from jax.experimental.pallas import tpu as pltpu, tpu_sc as plsc

N = 10000
NPAD = 10240
IN_D = 128
HID = 128
OUT_D = 64
E = 320000
NC = 2
NS = 16
NW = NC * NS
K = 128
EPW = 10240
EPAD = EPW * NW
EPWP = EPW + K
NPADH = NPAD // NC
NACC = NPADH + 128
RPT = NPADH // NS
LH = 16

_mesh = lambda: plsc.VectorSubcoreMesh(
    core_axis_name="c", subcore_axis_name="s", num_cores=NC, num_subcores=NS)


@functools.cache
def _make_prep():
    @functools.partial(
        pl.kernel,
        out_type=(
            jax.ShapeDtypeStruct((2, NW, NPAD), jnp.float32),
            jax.ShapeDtypeStruct((2, NW, EPWP), jnp.int32),
            jax.ShapeDtypeStruct((2, NW, EPWP), jnp.int32),
            jax.ShapeDtypeStruct((NW, 16), jnp.int32),
        ),
        mesh=_mesh(),
        compiler_params=pltpu.CompilerParams(needs_layout_passes=False),
        scratch_types=[
            pltpu.VMEM((EPW,), jnp.int32),
            pltpu.VMEM((EPW,), jnp.int32),
            pltpu.VMEM((NPAD,), jnp.float32),
            pltpu.VMEM((EPWP,), jnp.int32),
            pltpu.VMEM((EPWP,), jnp.int32),
            pltpu.VMEM((EPWP,), jnp.int32),
            pltpu.VMEM((EPWP,), jnp.int32),
            pltpu.VMEM((16,), jnp.int32),
        ],
    )
    def _prep_sc(src_hbm, dst_hbm, hist_hbm, psrc_hbm, pdst_hbm,
                 cnt_hbm, si_all, di_all, hist_v, ls_v, ld_v, hs_v, hd_v,
                 cnt_v):
        cid = lax.axis_index("c")
        sid = lax.axis_index("s")
        wid = sid * NC + cid
        ereal = E // NW
        base = pl.multiple_of(wid * ereal, 8)
        pltpu.sync_copy(src_hbm.at[pl.ds(base, ereal)], si_all.at[pl.ds(0, ereal)])
        pltpu.sync_copy(dst_hbm.at[pl.ds(base, ereal)], di_all.at[pl.ds(0, ereal)])
        iot16 = lax.iota(jnp.int32, 16)
        for j in range((EPW - ereal) // 16):
            pad = N + iot16 + (j * 16)
            si_all[pl.ds(ereal + j * 16, 16)] = pad
            di_all[pl.ds(ereal + j * 16, 16)] = pad
        ones = jnp.full((16,), 1.0, jnp.float32)
        zeros16 = jnp.zeros((16,), jnp.float32)

        def count(ref):
            def step(j, carry):
                plsc.addupdate_scatter(hist_v, [ref[pl.ds(j * 16, 16)]], ones)
                return carry
            return step

        def zero_hist(j, carry):
            hist_v[pl.ds(j * 16, 16)] = zeros16
            return carry

        lax.fori_loop(0, NPAD // 16, zero_hist, 0)
        lax.fori_loop(0, EPW // 16, count(si_all), 0)
        pltpu.sync_copy(hist_v, hist_hbm.at[0, wid])
        lax.fori_loop(0, NPAD // 16, zero_hist, 0)
        lax.fori_loop(0, EPW // 16, count(di_all), 0)
        pltpu.sync_copy(hist_v, hist_hbm.at[1, wid])

        def part(j, carry):
            nlo, nhi = carry
            sv = si_all[pl.ds(j * 16, 16)]
            dv = di_all[pl.ds(j * 16, 16)]
            m = dv < NPADH
            plsc.store_compressed(ls_v.at[pl.ds(nlo, 16)], sv, mask=m)
            plsc.store_compressed(ld_v.at[pl.ds(nlo, 16)], dv, mask=m)
            nm = jnp.logical_not(m)
            plsc.store_compressed(hs_v.at[pl.ds(nhi, 16)], sv, mask=nm)
            plsc.store_compressed(hd_v.at[pl.ds(nhi, 16)], dv, mask=nm)
            c = jnp.sum(m.astype(jnp.int32))
            return (nlo + c, nhi + 16 - c)

        nlo, nhi = lax.fori_loop(0, EPW // 16, part, (0, 0))

        iot16 = lax.iota(jnp.int32, 16)
        for j in range(K // 16):
            dummy = iot16 + (j * 16)
            dsrc = N + dummy
            ls_v[pl.ds(nlo + j * 16, 16)] = dsrc
            ld_v[pl.ds(nlo + j * 16, 16)] = NPADH + dummy
            hs_v[pl.ds(nhi + j * 16, 16)] = dsrc
            hd_v[pl.ds(nhi + j * 16, 16)] = 2 * NPADH + dummy
        pltpu.sync_copy(ls_v, psrc_hbm.at[0, wid])
        pltpu.sync_copy(ld_v, pdst_hbm.at[0, wid])
        pltpu.sync_copy(hs_v, psrc_hbm.at[1, wid])
        pltpu.sync_copy(hd_v, pdst_hbm.at[1, wid])
        iot = lax.iota(jnp.int32, 16)
        nloc = (nlo + K - 1) // K
        nhic = (nhi + K - 1) // K
        cnt_v[...] = jnp.where(iot == 0, nloc, jnp.where(iot == 1, nhic, 0))
        pltpu.sync_copy(cnt_v, cnt_hbm.at[wid])

    return _prep_sc


def _make_agg(D):
    @functools.partial(
        pl.kernel,
        out_type=jax.ShapeDtypeStruct((NC, NPADH, D), jnp.float32),
        mesh=_mesh(),
        compiler_params=pltpu.CompilerParams(needs_layout_passes=False),
        scratch_types=[
            pltpu.VMEM((K,), jnp.int32),
            pltpu.VMEM((K,), jnp.int32),
            pltpu.VMEM((K,), jnp.int32),
            pltpu.VMEM((K,), jnp.int32),
            pltpu.VMEM((K,), jnp.int32),
            pltpu.VMEM((K,), jnp.int32),
            pltpu.VMEM((K, D), jnp.float32),
            pltpu.VMEM((K, D), jnp.float32),
            pltpu.VMEM((RPT, D), jnp.float32),
            pltpu.VMEM((16,), jnp.int32),
            pltpu.SemaphoreType.DMA,
            pltpu.SemaphoreType.DMA,
            pltpu.SemaphoreType.DMA,
            pltpu.SemaphoreType.DMA,
            pltpu.VMEM_SHARED((NACC, D), jnp.float32),
        ],
    )
    def _agg_sc(z_hbm, psrc_hbm, pdst_hbm, cnt_hbm, out_hbm,
                si0, si1, di0, di1, dd0, dd1, rows0, rows1, buf, cnt_v,
                sem0, sem1, sem0b, sem1b, acc):
        si = [si0, si1]
        di = [di0, di1]
        dd = [dd0, dd1]
        rows = [rows0, rows1]
        sem = [sem0, sem1]
        semb = [sem0b, sem1b]
        cid = lax.axis_index("c")
        sid = lax.axis_index("s")
        r0 = sid * RPT
        lo = cid * NPADH
        zeros16 = jnp.zeros((16,), jnp.float32)

        def zero_row(rr, carry):
            for j in range(D // 16):
                buf[rr, pl.ds(j * 16, 16)] = zeros16
            return carry

        lax.fori_loop(0, RPT, zero_row, 0)
        pltpu.sync_copy(buf, acc.at[pl.ds(r0, RPT)])
        plsc.subcore_barrier()

        def region(r):
            pltpu.sync_copy(cnt_hbm.at[r], cnt_v)
            iot = lax.iota(jnp.int32, 16)
            nch = jnp.sum(jnp.where(iot == cid, cnt_v[...], 0))

            def load_idx(i, b):
                base = i * K
                pltpu.sync_copy(psrc_hbm.at[cid, r, pl.ds(base, K)], si[b])
                pltpu.sync_copy(pdst_hbm.at[cid, r, pl.ds(base, K)], di[b])

            H2 = K // 2

            def gather(b):
                pltpu.async_copy(z_hbm.at[si[b].at[pl.ds(0, H2)]],
                                 rows[b].at[pl.ds(0, H2)], sem[b])
                pltpu.async_copy(z_hbm.at[si[b].at[pl.ds(H2, H2)]],
                                 rows[b].at[pl.ds(H2, H2)], semb[b])

            def wait_gather(b):
                pltpu.make_async_copy(z_hbm.at[si[b].at[pl.ds(0, H2)]],
                                      rows[b].at[pl.ds(0, H2)], sem[b]).wait()
                pltpu.make_async_copy(z_hbm.at[si[b].at[pl.ds(H2, H2)]],
                                      rows[b].at[pl.ds(H2, H2)], semb[b]).wait()

            def consume(b):
                for j in range(K // 16):
                    dd[b][pl.ds(j * 16, 16)] = di[b][pl.ds(j * 16, 16)] - lo
                pltpu.sync_copy(rows[b], acc.at[dd[b]], add=True)

            @pl.when(nch > 0)
            def _():
                load_idx(0, 0)
                gather(0)

            def body(h, carry):
                i0 = 2 * h
                i1 = i0 + 1
                i2 = i0 + 2

                @pl.when(i0 < nch)
                def _():
                    wait_gather(0)
                    @pl.when(i1 < nch)
                    def _():
                        load_idx(i1, 1)
                        gather(1)
                    consume(0)

                @pl.when(i1 < nch)
                def _():
                    wait_gather(1)
                    @pl.when(i2 < nch)
                    def _():
                        load_idx(i2, 0)
                        gather(0)
                    consume(1)
                return carry

            lax.fori_loop(0, (nch + 1) // 2, body, 0)

        for t in range(2):
            region(2 * sid + t)
        plsc.subcore_barrier()
        pltpu.sync_copy(acc.at[pl.ds(r0, RPT)], buf)
        pltpu.sync_copy(buf, out_hbm.at[cid, pl.ds(r0, RPT)])

    return _agg_sc


_make_agg = functools.cache(_make_agg)


def _deg_norms(hp):
    degs = jnp.sum(hp[0], axis=0)
    degd = jnp.sum(hp[1], axis=0)
    ns = lax.rsqrt(jnp.maximum(degs, 1.0))
    nd = lax.rsqrt(jnp.maximum(degd, 1.0))
    return ns, nd


def _z1_body(x_ref, w_ref, h_ref, o_ref):
    ns, _ = _deg_norms(h_ref[...])
    xw = jnp.dot(x_ref[...], w_ref[...], preferred_element_type=jnp.float32)
    o_ref[0:N, :] = xw * ns[0:N, None]
    o_ref[N:NPAD, :] = jnp.zeros((NPAD - N, HID), jnp.float32)


def _mid_body(acc_ref, h_ref, b1_ref, g1_ref, be1_ref, w2_ref, o_ref):
    ns, nd = _deg_norms(h_ref[...])
    a = acc_ref[...]
    h = a * nd[:, None] + b1_ref[...][None, :]
    mask = (lax.broadcasted_iota(jnp.int32, (NPAD, 1), 0) < N).astype(jnp.float32)
    mean = jnp.sum(h * mask, axis=0) / N
    cen = h - mean[None, :]
    var = jnp.sum(cen * cen * mask, axis=0) / N
    hbn = cen * lax.rsqrt(var + 1e-5)[None, :] * g1_ref[...][None, :] + be1_ref[...][None, :]
    hr = jnp.maximum(hbn, 0.0)
    z2 = jnp.dot(hr, w2_ref[...], preferred_element_type=jnp.float32)
    o_ref[...] = z2 * ns[:, None] * mask


def _out_body(acc_ref, h_ref, b2_ref, g2_ref, be2_ref, o_ref):
    _, nd = _deg_norms(h_ref[...])
    a = acc_ref[...][:, :OUT_D]
    h = a * nd[:, None] + b2_ref[...][None, :]
    mask = (lax.broadcasted_iota(jnp.int32, (NPAD, 1), 0) < N).astype(jnp.float32)
    mean = jnp.sum(h * mask, axis=0) / N
    cen = h - mean[None, :]
    var = jnp.sum(cen * cen * mask, axis=0) / N
    hbn = cen * lax.rsqrt(var + 1e-5)[None, :] * g2_ref[...][None, :] + be2_ref[...][None, :]
    hh = hbn[:N, :]
    m = jnp.max(hh, axis=1, keepdims=True)
    ex = jnp.exp(hh - m)
    lse = jnp.log(jnp.sum(ex, axis=1, keepdims=True))
    o_ref[...] = hh - m - lse


def _tc_call(body, out_shape, *args):
    return pl.pallas_call(
        body, out_shape=jax.ShapeDtypeStruct(out_shape, jnp.float32))(*args)


def kernel(x, W1, b1, gamma1, beta1, W2, b2, gamma2, beta2, edge_index):
    src = edge_index[0].astype(jnp.int32)
    dst = edge_index[1].astype(jnp.int32)
    W2p = jnp.pad(W2, ((0, 0), (0, HID - OUT_D)))

    histp, psrc, pdst, cnt = _make_prep()(src, dst)
    z1 = _tc_call(_z1_body, (NPAD, HID), x, W1, histp)
    agg = _make_agg(HID)
    acc1 = agg(z1, psrc, pdst, cnt)
    z2 = _tc_call(_mid_body, (NPAD, HID), acc1.reshape(NPAD, HID), histp, b1,
                  gamma1, beta1, W2p)
    acc2 = agg(z2, psrc, pdst, cnt)
    out = _tc_call(_out_body, (N, OUT_D), acc2.reshape(NPAD, HID), histp, b2,
                  gamma2, beta2)
    return out
